# Initial kernel scaffold; baseline (speedup 1.0000x reference)
import jax
import jax.numpy as jnp
from jax.experimental import pallas as pl


def kernel(x, edge_index, batch, W0, b0, W1, b1, W2, b2):
    raise NotImplementedError("write your pallas kernel here")


# speedup vs baseline: 28.2790x; 28.2790x over previous
import functools

import jax
import jax.numpy as jnp
from jax import lax
from jax.experimental import pallas as pl
from jax.experimental.pallas import tpu as pltpu
from jax.experimental.pallas import tpu_sc as plsc

N = 10000
D = 128
E = 320000
G = 64
NP = 10240
NC = 2
NS = 16
NW = NC * NS
EPT = 10368
EP = NW * EPT
KE = 32
CE = EPT // KE
K = 128
C = EPT // K
GAG = 4
GAS = 2
NSL = GAG + GAS
IA = 8
NIX = IA + GAG
NPT = NP // NS
R = 1024
GRID = NP // R

assert EPT % KE == 0 and EPT % K == 0 and NPT % KE == 0
assert NIX % NSL == 0 and CE % NIX == 0 and NSL == GAG + GAS and IA > GAG


def _leaky(u):
    return jnp.where(u >= 0, u, 0.01 * u)


def _dis_of(dg):
    return lax.rsqrt(dg[:, 0:1] + dg[:, 1:2] + 1.0)


_MESH = plsc.VectorSubcoreMesh(
    core_axis_name="c", subcore_axis_name="s", num_cores=NC, num_subcores=NS
)


@functools.partial(
    pl.kernel,
    out_type=jax.ShapeDtypeStruct((NC, NP), jnp.float32),
    mesh=_MESH,
    scratch_types=[
        pltpu.VMEM((C, K), jnp.int32),
        pltpu.VMEM((K,), jnp.float32),
        pltpu.VMEM((NPT,), jnp.float32),
        pltpu.VMEM_SHARED((NP,), jnp.float32),
    ],
)
def _deg_kernel(dst_hbm, out_hbm, dstv, onev, zrow, acc):
    cid = lax.axis_index("c")
    sid = lax.axis_index("s")
    wid = sid * NC + cid

    def fill(i, c):
        zrow[pl.ds(i * 16, 16)] = jnp.zeros((16,), jnp.float32)
        return c

    lax.fori_loop(0, NPT // 16, fill, 0)

    def fill1(i, c):
        onev[pl.ds(i * 16, 16)] = jnp.ones((16,), jnp.float32)
        return c

    lax.fori_loop(0, K // 16, fill1, 0)

    base = sid * NPT
    pltpu.sync_copy(zrow, acc.at[pl.ds(base, NPT)])
    pltpu.sync_copy(dst_hbm.at[wid], dstv)
    plsc.subcore_barrier()

    def body(ci, c):
        pltpu.sync_copy(onev, acc.at[dstv.at[ci]], add=True)
        return c

    lax.fori_loop(0, C, body, 0)
    plsc.subcore_barrier()
    pltpu.sync_copy(acc.at[pl.ds(base, NPT)], out_hbm.at[cid, pl.ds(base, NPT)])


@functools.partial(
    pl.kernel,
    out_type=jax.ShapeDtypeStruct((NC, NP, D), jnp.float32),
    mesh=_MESH,
    scratch_types=[
        [pltpu.VMEM((KE,), jnp.int32) for _ in range(NIX)],
        [pltpu.VMEM((KE,), jnp.int32) for _ in range(NIX)],
        [pltpu.VMEM((KE, D), jnp.float32) for _ in range(NSL)],
        pltpu.VMEM_SHARED((NP, D), jnp.float32),
        [pltpu.SemaphoreType.DMA for _ in range(NSL)],
        [pltpu.SemaphoreType.DMA for _ in range(NSL)],
        [pltpu.SemaphoreType.DMA for _ in range(NIX)],
    ],
)
def _edge_kernel(t_hbm, src_hbm, dst_hbm, out_hbm, sidx, didx, rows, acc,
                 gsem, ssem, isem):
    cid = lax.axis_index("c")
    sid = lax.axis_index("s")
    wid = sid * NC + cid

    def zb(i, c):
        for b in range(NSL):
            rows[b][i // 8, pl.ds(lax.rem(i, 8) * 16, 16)] = jnp.zeros(
                (16,), jnp.float32
            )
        return c

    lax.fori_loop(0, KE * (D // 16), zb, 0)

    base = sid * NPT
    NZ = NPT // KE

    def zbody(j, c):
        for b in range(NSL):
            k = j * NSL + b

            @pl.when((k >= NSL) & (k < NZ + NSL))
            def _(b=b):
                pltpu.make_async_copy(
                    rows[b], acc.at[pl.ds(base, KE)], gsem[b]
                ).wait()

            @pl.when(k < NZ)
            def _(b=b, k=k):
                pltpu.async_copy(
                    rows[b], acc.at[pl.ds(base + k * KE, KE)], gsem[b]
                )

        return c

    lax.fori_loop(0, (NZ + 2 * NSL - 1) // NSL, zbody, 0)
    plsc.subcore_barrier()

    def ifetch(ci, bi):
        pltpu.async_copy(src_hbm.at[wid, ci], sidx[bi], isem[bi])
        pltpu.async_copy(dst_hbm.at[wid, ci], didx[bi], isem[bi])

    def iwait(ci, bi):
        pltpu.make_async_copy(src_hbm.at[wid, ci], sidx[bi], isem[bi]).wait()
        pltpu.make_async_copy(dst_hbm.at[wid, ci], didx[bi], isem[bi]).wait()

    def gath(ci, b, bi):
        pltpu.async_copy(t_hbm.at[sidx[bi]], rows[b], gsem[b])

    def gath_wait(ci, b, bi):
        pltpu.make_async_copy(t_hbm.at[sidx[bi]], rows[b], gsem[b]).wait()

    def scat(ci, b, bi):
        pltpu.async_copy(rows[b], acc.at[didx[bi]], ssem[b], add=True)

    def scat_wait(ci, b, bi):
        pltpu.make_async_copy(rows[b], acc.at[didx[bi]], ssem[b]).wait()

    for ci in range(IA):
        ifetch(ci, ci)
    for ci in range(GAG):
        iwait(ci, ci)
        gath(ci, ci, ci)

    def visit(c, b, bi):
        gath_wait(c, b, bi)
        scat(c, b, bi)

        @pl.when(c >= GAS)
        def _():
            scat_wait(c - GAS, (b + NSL - GAS) % NSL, (bi + NIX - GAS) % NIX)

        @pl.when(c + IA < CE)
        def _():
            ifetch(c + IA, (bi + IA) % NIX)

        @pl.when(c + GAG < CE)
        def _():
            iwait(c + GAG, (bi + GAG) % NIX)
            gath(c + GAG, (b + GAG) % NSL, (bi + GAG) % NIX)

    def body(j, c):
        c0 = j * NIX
        for k in range(NIX):
            visit(c0 + k, k % NSL, k % NIX)
        return c

    lax.fori_loop(0, CE // NIX, body, 0)
    for i in range(GAS):
        scat_wait(CE - GAS + i, (CE - GAS + i) % NSL, (CE - GAS + i) % NIX)
    plsc.subcore_barrier()
    pltpu.sync_copy(acc.at[pl.ds(base, NPT)], out_hbm.at[cid, pl.ds(base, NPT)])


def _mm1_body(x_ref, w_ref, dg_ref, o_ref):
    dis = _dis_of(dg_ref[...])
    o_ref[...] = dis * jnp.dot(
        x_ref[...], w_ref[...], preferred_element_type=jnp.float32
    )


def _mm1(xp, W, degt):
    return pl.pallas_call(
        _mm1_body,
        grid=(GRID,),
        in_specs=[
            pl.BlockSpec((R, D), lambda i: (i, 0)),
            pl.BlockSpec((D, D), lambda i: (0, 0)),
            pl.BlockSpec((R, 2), lambda i: (i, 0)),
        ],
        out_specs=pl.BlockSpec((R, D), lambda i: (i, 0)),
        out_shape=jax.ShapeDtypeStruct((NP, D), jnp.float32),
    )(xp, W, degt)


def _mmf_body(has_res, p0_ref, p1_ref, t_ref, dg_ref, b_ref, *rest):
    if has_res:
        r_ref, w_ref, tn_ref, h_ref = rest
    else:
        w_ref, tn_ref, h_ref = rest
    dis = _dis_of(dg_ref[...])
    u = dis * (p0_ref[...] + p1_ref[...] + t_ref[...]) + b_ref[...]
    if has_res:
        u = u + r_ref[...]
    h = _leaky(u)
    h_ref[...] = h
    tn_ref[...] = dis * jnp.dot(h, w_ref[...], preferred_element_type=jnp.float32)


def _mm_fused(p0, p1, t, degt, b2d, res, W):
    has_res = res is not None
    specs = [
        pl.BlockSpec((R, D), lambda i: (i, 0)),
        pl.BlockSpec((R, D), lambda i: (i, 0)),
        pl.BlockSpec((R, D), lambda i: (i, 0)),
        pl.BlockSpec((R, 2), lambda i: (i, 0)),
        pl.BlockSpec((1, D), lambda i: (0, 0)),
    ]
    args = [p0, p1, t, degt, b2d]
    if has_res:
        specs.append(pl.BlockSpec((R, D), lambda i: (i, 0)))
        args.append(res)
    specs.append(pl.BlockSpec((D, D), lambda i: (0, 0)))
    args.append(W)
    return pl.pallas_call(
        functools.partial(_mmf_body, has_res),
        grid=(GRID,),
        in_specs=specs,
        out_specs=[
            pl.BlockSpec((R, D), lambda i: (i, 0)),
            pl.BlockSpec((R, D), lambda i: (i, 0)),
        ],
        out_shape=[
            jax.ShapeDtypeStruct((NP, D), jnp.float32),
            jax.ShapeDtypeStruct((NP, D), jnp.float32),
        ],
    )(*args)


def _pool_body(p0_ref, p1_ref, t_ref, dg_ref, b_ref, r_ref, bt_ref, o_ref,
               sums, counts):
    i = pl.program_id(0)

    @pl.when(i == 0)
    def _():
        sums[...] = jnp.zeros_like(sums)
        counts[...] = jnp.zeros_like(counts)

    dis = _dis_of(dg_ref[...])
    u = dis * (p0_ref[...] + p1_ref[...] + t_ref[...]) + b_ref[...] + r_ref[...]
    h = _leaky(u)
    onehot = (
        bt_ref[...] == lax.broadcasted_iota(jnp.int32, (R, G), 1)
    ).astype(jnp.float32)
    dn = (((0,), (0,)), ((), ()))
    sums[...] += lax.dot_general(onehot, h, dn, preferred_element_type=jnp.float32)
    counts[...] += lax.dot_general(
        onehot, jnp.ones((R, D), jnp.float32), dn,
        preferred_element_type=jnp.float32,
    )

    @pl.when(i == pl.num_programs(0) - 1)
    def _():
        o_ref[...] = sums[...] / jnp.maximum(counts[...], 1.0)


def _pool(p0, p1, t, degt, b2d, res, bt2d):
    return pl.pallas_call(
        _pool_body,
        grid=(GRID,),
        in_specs=[
            pl.BlockSpec((R, D), lambda i: (i, 0)),
            pl.BlockSpec((R, D), lambda i: (i, 0)),
            pl.BlockSpec((R, D), lambda i: (i, 0)),
            pl.BlockSpec((R, 2), lambda i: (i, 0)),
            pl.BlockSpec((1, D), lambda i: (0, 0)),
            pl.BlockSpec((R, D), lambda i: (i, 0)),
            pl.BlockSpec((R, 1), lambda i: (i, 0)),
        ],
        out_specs=pl.BlockSpec((G, D), lambda i: (0, 0)),
        out_shape=jax.ShapeDtypeStruct((G, D), jnp.float32),
        scratch_shapes=[
            pltpu.VMEM((G, D), jnp.float32),
            pltpu.VMEM((G, D), jnp.float32),
        ],
    )(p0, p1, t, degt, b2d, res, bt2d)


def kernel(x, edge_index, batch, W0, b0, W1, b1, W2, b2):
    xp = jnp.pad(x, ((0, NP - N), (0, 0)))
    pad_idx = N + (jnp.arange(EP - E, dtype=jnp.int32) % (NP - N))
    src_f = jnp.concatenate([edge_index[0], pad_idx])
    dst_f = jnp.concatenate([edge_index[1], pad_idx])
    src_r = src_f.reshape(NW, CE, KE)
    dst_r = dst_f.reshape(NW, CE, KE)
    dst_d = dst_f.reshape(NW, C, K)
    bt2d = jnp.concatenate(
        [batch, jnp.full((NP - N,), -1, jnp.int32)]
    ).reshape(NP, 1)
    b0r, b1r, b2r = (b.reshape(1, D) for b in (b0, b1, b2))

    degp = _deg_kernel(dst_d)
    degt = degp.T

    t1 = _mm1(xp, W0, degt)
    p1 = _edge_kernel(t1, src_r, dst_r)
    t2, h1 = _mm_fused(p1[0], p1[1], t1, degt, b0r, None, W1)
    p2 = _edge_kernel(t2, src_r, dst_r)
    t3, h2 = _mm_fused(p2[0], p2[1], t2, degt, b1r, h1, W2)
    p3 = _edge_kernel(t3, src_r, dst_r)
    return _pool(p3[0], p3[1], t3, degt, b2r, h2, bt2d)
